# Initial kernel scaffold; baseline (speedup 1.0000x reference)
import jax
import jax.numpy as jnp
from jax.experimental import pallas as pl


def kernel(x, edge_index, W1, b1, W2, b2, Wf, bf):
    raise NotImplementedError("write your pallas kernel here")


# speedup vs baseline: 58.0089x; 58.0089x over previous
import functools

import jax
import jax.numpy as jnp
from jax import lax
from jax.experimental import pallas as pl
from jax.experimental.pallas import tpu as pltpu
from jax.experimental.pallas import tpu_sc as plsc

N = 100000
E = 3200000
NC, NS = 2, 16
NW = NC * NS
GROUP = 1024
SUB = 128
NSUB = GROUP // SUB
EP = ((E + NW * GROUP - 1) // (NW * GROUP)) * (NW * GROUP)
EPW = EP // NW
G = EPW // GROUP
NP = 100352
DRAIN = NP // NS
PADROWS = NP - N
BN = 2048
NB = NP // BN

def _chunks(total, size):
    out, off = [], 0
    while off < total:
        c = min(size, total - off)
        out.append((off, c))
        off += c
    return out


def _deg_body(dst_hbm, out0, out1, didx, vals, acc, sem_in, sem_s):
    c = lax.axis_index("c")
    s = lax.axis_index("s")
    wid = s * NC + c
    base = wid * EPW
    tile0 = s * DRAIN

    @pl.loop(0, GROUP)
    def _(i):
        vals[i, :] = jnp.zeros((16,), jnp.float32)

    for off, ln in _chunks(DRAIN, GROUP):
        pltpu.sync_copy(vals.at[pl.ds(0, ln), :],
                        acc.at[pl.ds(tile0 + off, ln), :])

    @pl.loop(0, SUB)
    def _(i):
        vals[i, :] = jnp.ones((16,), jnp.float32)

    plsc.subcore_barrier()

    @pl.loop(0, G)
    def _(g):
        e0 = base + g * GROUP
        ds_in = [
            pltpu.async_copy(dst_hbm.at[pl.ds(e0 + j * SUB, SUB)],
                             didx.at[j], sem_in)
            for j in range(NSUB)
        ]
        for d in ds_in:
            d.wait()
        ds_s = [
            pltpu.async_copy(vals.at[pl.ds(0, SUB), :],
                             acc.at[didx.at[j]], sem_s, add=True)
            for j in range(NSUB)
        ]
        for d in ds_s:
            d.wait()

    plsc.subcore_barrier()

    @pl.when(c == 0)
    def _():
        pltpu.sync_copy(acc.at[pl.ds(tile0, DRAIN), :],
                        out0.at[pl.ds(tile0, DRAIN), :])

    @pl.when(c == 1)
    def _():
        pltpu.sync_copy(acc.at[pl.ds(tile0, DRAIN), :],
                        out1.at[pl.ds(tile0, DRAIN), :])


def _prop_body(table, src_hbm, dst_hbm, out0, out1,
               sidx, didx, rows, acc, sem_in, sem_g, sem_s):
    c = lax.axis_index("c")
    s = lax.axis_index("s")
    wid = s * NC + c
    base = wid * EPW
    tile0 = s * DRAIN

    @pl.loop(0, GROUP)
    def _(i):
        rows[i, :] = jnp.zeros((16,), jnp.float32)

    for off, ln in _chunks(DRAIN, GROUP):
        pltpu.sync_copy(rows.at[pl.ds(0, ln), :],
                        acc.at[pl.ds(tile0 + off, ln), :])
    plsc.subcore_barrier()

    @pl.loop(0, G)
    def _(g):
        e0 = base + g * GROUP
        ds_in = [pltpu.async_copy(src_hbm.at[pl.ds(e0, GROUP)], sidx, sem_in)]
        ds_in += [
            pltpu.async_copy(dst_hbm.at[pl.ds(e0 + j * SUB, SUB)],
                             didx.at[j], sem_in)
            for j in range(NSUB)
        ]
        for d in ds_in:
            d.wait()
        ds_g = [
            pltpu.async_copy(table.at[sidx.at[pl.ds(j * SUB, SUB)]],
                             rows.at[pl.ds(j * SUB, SUB), :], sem_g)
            for j in range(NSUB)
        ]
        for d in ds_g:
            d.wait()
        ds_s = [
            pltpu.async_copy(rows.at[pl.ds(j * SUB, SUB), :],
                             acc.at[didx.at[j]], sem_s, add=True)
            for j in range(NSUB)
        ]
        for d in ds_s:
            d.wait()

    plsc.subcore_barrier()

    @pl.when(c == 0)
    def _():
        pltpu.sync_copy(acc.at[pl.ds(tile0, DRAIN), :],
                        out0.at[pl.ds(tile0, DRAIN), :])

    @pl.when(c == 1)
    def _():
        pltpu.sync_copy(acc.at[pl.ds(tile0, DRAIN), :],
                        out1.at[pl.ds(tile0, DRAIN), :])


@functools.lru_cache(maxsize=None)
def _sc_kernels():
    mesh = plsc.VectorSubcoreMesh(core_axis_name="c", subcore_axis_name="s",
                                  num_cores=NC, num_subcores=NS)
    params = pltpu.CompilerParams(use_tc_tiling_on_sc=False)
    deg = pl.kernel(
        _deg_body,
        compiler_params=params,
        out_type=[jax.ShapeDtypeStruct((NP, 16), jnp.float32),
                  jax.ShapeDtypeStruct((NP, 16), jnp.float32)],
        mesh=mesh,
        scratch_types=[
            pltpu.VMEM((NSUB, SUB), jnp.int32),
            pltpu.VMEM((GROUP, 16), jnp.float32),
            pltpu.VMEM_SHARED((NP, 16), jnp.float32),
            pltpu.SemaphoreType.DMA,
            pltpu.SemaphoreType.DMA,
        ],
    )
    prop = pl.kernel(
        _prop_body,
        compiler_params=params,
        out_type=[jax.ShapeDtypeStruct((NP, 16), jnp.float32),
                  jax.ShapeDtypeStruct((NP, 16), jnp.float32)],
        mesh=mesh,
        scratch_types=[
            pltpu.VMEM((GROUP,), jnp.int32),
            pltpu.VMEM((NSUB, SUB), jnp.int32),
            pltpu.VMEM((GROUP, 16), jnp.float32),
            pltpu.VMEM_SHARED((NP, 16), jnp.float32),
            pltpu.SemaphoreType.DMA,
            pltpu.SemaphoreType.DMA,
            pltpu.SemaphoreType.DMA,
        ],
    )
    return deg, prop


def _tc1_body(d0_ref, d1_ref, x_ref, t1_ref):
    deg = d0_ref[...] + d1_ref[...] + 1.0
    dis = lax.rsqrt(deg)
    t1_ref[...] = x_ref[...] * dis


def _tc1(deg0, deg1, x16):
    blk = pl.BlockSpec((BN, 16), lambda i: (i, 0))
    return pl.pallas_call(
        _tc1_body,
        grid=(NB,),
        in_specs=[blk, blk, blk],
        out_specs=blk,
        out_shape=jax.ShapeDtypeStruct((NP, 16), jnp.float32),
    )(deg0, deg1, x16)


def _tc2_body(t1_ref, a_ref, b_ref, w1_ref, b1_ref, w2_ref, t2_ref):
    t1 = t1_ref[...]
    dis = t1[:, 15:16]
    stot = t1 + a_ref[...] + b_ref[...]
    h1 = jnp.dot(stot, w1_ref[...], preferred_element_type=jnp.float32)
    out1 = jnp.maximum(h1 * dis + b1_ref[...], 0.0)
    h2 = jnp.dot(out1, w2_ref[...], preferred_element_type=jnp.float32)
    t2_ref[...] = h2 * dis


def _tc2(t1, s1a, s1b, w1p, b1, w2):
    blk = pl.BlockSpec((BN, 16), lambda i: (i, 0))
    return pl.pallas_call(
        _tc2_body,
        grid=(NB,),
        in_specs=[
            blk, blk, blk,
            pl.BlockSpec((16, 32), lambda i: (0, 0)),
            pl.BlockSpec((1, 32), lambda i: (0, 0)),
            pl.BlockSpec((32, 16), lambda i: (0, 0)),
        ],
        out_specs=pl.BlockSpec((BN, 16), lambda i: (i, 0)),
        out_shape=jax.ShapeDtypeStruct((NP, 16), jnp.float32),
    )(t1, s1a, s1b, w1p, b1.reshape(1, 32), w2)


def _tc3_body(t2_ref, a_ref, b_ref, d0_ref, d1_ref, b2_ref, wf_ref, bf_ref,
              y_ref):
    deg = d0_ref[...] + d1_ref[...] + 1.0
    dis = lax.rsqrt(deg)
    stot = t2_ref[...] + a_ref[...] + b_ref[...]
    out2 = jnp.maximum(stot * dis + b2_ref[...], 0.0)
    y_ref[...] = jnp.dot(out2, wf_ref[...],
                         preferred_element_type=jnp.float32) + bf_ref[...]


def _tc3(t2, s2a, s2b, deg0, deg1, b2, wf, bf):
    blk = pl.BlockSpec((BN, 16), lambda i: (i, 0))
    return pl.pallas_call(
        _tc3_body,
        grid=(NB,),
        in_specs=[
            blk, blk, blk, blk, blk,
            pl.BlockSpec((1, 16), lambda i: (0, 0)),
            pl.BlockSpec((16, 1), lambda i: (0, 0)),
            pl.BlockSpec((1, 1), lambda i: (0, 0)),
        ],
        out_specs=pl.BlockSpec((BN, 1), lambda i: (i, 0)),
        out_shape=jax.ShapeDtypeStruct((N, 1), jnp.float32),
    )(t2, s2a, s2b, deg0, deg1, b2.reshape(1, 16), wf, bf.reshape(1, 1))


def kernel(x, edge_index, W1, b1, W2, b2, Wf, bf):
    src = edge_index[0].astype(jnp.int32)
    dst = edge_index[1].astype(jnp.int32)
    npad = EP - E
    pad_idx = N + (jnp.arange(npad, dtype=jnp.int32) % PADROWS)
    src_p = jnp.concatenate([src, pad_idx])
    dst_p = jnp.concatenate([dst, pad_idx])

    x16 = jnp.zeros((NP, 16), jnp.float32)
    x16 = x16.at[:N, :3].set(x.astype(jnp.float32))
    x16 = x16.at[:, 15].set(1.0)

    w1p = jnp.zeros((16, 32), jnp.float32).at[:3, :].set(W1)

    deg_k, prop_k = _sc_kernels()
    deg0, deg1 = deg_k(dst_p)
    t1 = _tc1(deg0, deg1, x16)
    s1a, s1b = prop_k(t1, src_p, dst_p)
    t2 = _tc2(t1, s1a, s1b, w1p, b1, W2)
    s2a, s2b = prop_k(t2, src_p, dst_p)
    return _tc3(t2, s2a, s2b, deg0, deg1, b2, Wf, bf)


# speedup vs baseline: 62.9908x; 1.0859x over previous
import functools

import jax
import jax.numpy as jnp
from jax import lax
from jax.experimental import pallas as pl
from jax.experimental.pallas import tpu as pltpu
from jax.experimental.pallas import tpu_sc as plsc

N = 100000
E = 3200000
NC, NS = 2, 16
NW = NC * NS
GROUP = 512
SUB = 128
NSUB = GROUP // SUB
EP = ((E + NW * GROUP - 1) // (NW * GROUP)) * (NW * GROUP)
EPW = EP // NW
G = EPW // GROUP
NP = 100352
DRAIN = NP // NS
PADROWS = NP - N
BN = 2048
NB = NP // BN

def _chunks(total, size):
    out, off = [], 0
    while off < total:
        c = min(size, total - off)
        out.append((off, c))
        off += c
    return out


def _deg_body(dst_hbm, out0, out1, didx, vals, acc,
              sem_in0, sem_in1, sem_s0, sem_s1):
    c = lax.axis_index("c")
    s = lax.axis_index("s")
    wid = s * NC + c
    base = wid * EPW
    tile0 = s * DRAIN
    sem_in = (sem_in0, sem_in1)
    sem_s = (sem_s0, sem_s1)

    @pl.loop(0, GROUP)
    def _(i):
        vals[i, :] = jnp.zeros((16,), jnp.float32)

    for off, ln in _chunks(DRAIN, GROUP):
        pltpu.sync_copy(vals.at[pl.ds(0, ln), :],
                        acc.at[pl.ds(tile0 + off, ln), :])

    @pl.loop(0, SUB)
    def _(i):
        vals[i, :] = jnp.ones((16,), jnp.float32)

    plsc.subcore_barrier()

    def in_descs(g, sl):
        e0 = (base + g * GROUP) % EP
        return [pltpu.make_async_copy(dst_hbm.at[pl.ds(e0 + j * SUB, SUB)],
                                      didx.at[sl, j], sem_in[sl])
                for j in range(NSUB)]

    def fire_sc(sl):
        for j in range(NSUB):
            pltpu.async_copy(vals.at[pl.ds(0, SUB), :],
                             acc.at[didx.at[sl, j]], sem_s[sl], add=True)

    def wait_sc(sl):
        for j in range(NSUB):
            pltpu.make_async_copy(vals.at[pl.ds(0, SUB), :],
                                  acc.at[didx.at[sl, j]], sem_s[sl]).wait()

    def fire(descs):
        for d in descs:
            d.start()

    def wait(descs):
        for d in descs:
            d.wait()

    def body(g, first):
        wait(in_descs(g, 0))
        if not first:
            wait_sc(1)
        fire(in_descs(g + 1, 1))
        fire_sc(0)
        wait(in_descs(g + 1, 1))
        wait_sc(0)
        fire(in_descs(g + 2, 0))
        fire_sc(1)

    fire(in_descs(0, 0))
    body(0, True)

    @pl.loop(2, G, step=2)
    def _(g):
        body(g, False)

    wait_sc(1)
    wait(in_descs(G, 0))

    plsc.subcore_barrier()

    @pl.when(c == 0)
    def _():
        pltpu.sync_copy(acc.at[pl.ds(tile0, DRAIN), :],
                        out0.at[pl.ds(tile0, DRAIN), :])

    @pl.when(c == 1)
    def _():
        pltpu.sync_copy(acc.at[pl.ds(tile0, DRAIN), :],
                        out1.at[pl.ds(tile0, DRAIN), :])


def _prop_body(table, src_hbm, dst_hbm, out0, out1,
               sidx, didx, rows, acc, sem_in0, sem_in1, sem_g0, sem_g1,
               sem_s0, sem_s1):
    c = lax.axis_index("c")
    s = lax.axis_index("s")
    wid = s * NC + c
    base = wid * EPW
    tile0 = s * DRAIN
    sem_in = (sem_in0, sem_in1)
    sem_g = (sem_g0, sem_g1)
    sem_s = (sem_s0, sem_s1)

    @pl.loop(0, GROUP)
    def _(i):
        rows[0, i, :] = jnp.zeros((16,), jnp.float32)

    for off, ln in _chunks(DRAIN, GROUP):
        pltpu.sync_copy(rows.at[0, pl.ds(0, ln), :],
                        acc.at[pl.ds(tile0 + off, ln), :])
    plsc.subcore_barrier()

    def in_descs(g, sl):
        e0 = (base + g * GROUP) % EP
        d = [pltpu.make_async_copy(src_hbm.at[pl.ds(e0, GROUP)],
                                   sidx.at[sl], sem_in[sl])]
        d += [pltpu.make_async_copy(dst_hbm.at[pl.ds(e0 + j * SUB, SUB)],
                                    didx.at[sl, j], sem_in[sl])
              for j in range(NSUB)]
        return d

    def ga_descs(g, sl):
        return [pltpu.make_async_copy(
            table.at[sidx.at[sl].at[pl.ds(j * SUB, SUB)]],
            rows.at[sl, pl.ds(j * SUB, SUB), :], sem_g[sl])
            for j in range(NSUB)]

    def fire_sc(g, sl):
        for j in range(NSUB):
            pltpu.async_copy(rows.at[sl, pl.ds(j * SUB, SUB), :],
                             acc.at[didx.at[sl, j]], sem_s[sl], add=True)

    def wait_sc(g, sl):
        for j in range(NSUB):
            pltpu.make_async_copy(rows.at[sl, pl.ds(j * SUB, SUB), :],
                                  acc.at[didx.at[sl, j]], sem_s[sl]).wait()

    def fire(descs):
        for d in descs:
            d.start()

    def wait(descs):
        for d in descs:
            d.wait()

    def body(g, first):
        wait(in_descs(g, 0))
        fire(ga_descs(g, 0))
        if not first:
            wait_sc(g - 1, 1)
        fire(in_descs(g + 1, 1))
        wait(ga_descs(g, 0))
        fire_sc(g, 0)
        wait(in_descs(g + 1, 1))
        fire(ga_descs(g + 1, 1))
        wait_sc(g, 0)
        fire(in_descs(g + 2, 0))
        wait(ga_descs(g + 1, 1))
        fire_sc(g + 1, 1)

    fire(in_descs(0, 0))
    body(0, True)

    @pl.loop(2, G, step=2)
    def _(g):
        body(g, False)

    wait_sc(G - 1, 1)
    wait(in_descs(G, 0))

    plsc.subcore_barrier()

    @pl.when(c == 0)
    def _():
        pltpu.sync_copy(acc.at[pl.ds(tile0, DRAIN), :],
                        out0.at[pl.ds(tile0, DRAIN), :])

    @pl.when(c == 1)
    def _():
        pltpu.sync_copy(acc.at[pl.ds(tile0, DRAIN), :],
                        out1.at[pl.ds(tile0, DRAIN), :])


@functools.lru_cache(maxsize=None)
def _sc_kernels():
    mesh = plsc.VectorSubcoreMesh(core_axis_name="c", subcore_axis_name="s",
                                  num_cores=NC, num_subcores=NS)
    params = pltpu.CompilerParams(use_tc_tiling_on_sc=False)
    deg = pl.kernel(
        _deg_body,
        compiler_params=params,
        out_type=[jax.ShapeDtypeStruct((NP, 16), jnp.float32),
                  jax.ShapeDtypeStruct((NP, 16), jnp.float32)],
        mesh=mesh,
        scratch_types=[
            pltpu.VMEM((2, NSUB, SUB), jnp.int32),
            pltpu.VMEM((GROUP, 16), jnp.float32),
            pltpu.VMEM_SHARED((NP, 16), jnp.float32),
            pltpu.SemaphoreType.DMA,
            pltpu.SemaphoreType.DMA,
            pltpu.SemaphoreType.DMA,
            pltpu.SemaphoreType.DMA,
        ],
    )
    prop = pl.kernel(
        _prop_body,
        compiler_params=params,
        out_type=[jax.ShapeDtypeStruct((NP, 16), jnp.float32),
                  jax.ShapeDtypeStruct((NP, 16), jnp.float32)],
        mesh=mesh,
        scratch_types=[
            pltpu.VMEM((2, GROUP), jnp.int32),
            pltpu.VMEM((2, NSUB, SUB), jnp.int32),
            pltpu.VMEM((2, GROUP, 16), jnp.float32),
            pltpu.VMEM_SHARED((NP, 16), jnp.float32),
            pltpu.SemaphoreType.DMA,
            pltpu.SemaphoreType.DMA,
            pltpu.SemaphoreType.DMA,
            pltpu.SemaphoreType.DMA,
            pltpu.SemaphoreType.DMA,
            pltpu.SemaphoreType.DMA,
        ],
    )
    return deg, prop


def _tc1_body(d0_ref, d1_ref, x_ref, t1_ref):
    deg = d0_ref[...] + d1_ref[...] + 1.0
    dis = lax.rsqrt(deg)
    t1_ref[...] = x_ref[...] * dis


def _tc1(deg0, deg1, x16):
    blk = pl.BlockSpec((BN, 16), lambda i: (i, 0))
    return pl.pallas_call(
        _tc1_body,
        grid=(NB,),
        in_specs=[blk, blk, blk],
        out_specs=blk,
        out_shape=jax.ShapeDtypeStruct((NP, 16), jnp.float32),
    )(deg0, deg1, x16)


def _tc2_body(t1_ref, a_ref, b_ref, w1_ref, b1_ref, w2_ref, t2_ref):
    t1 = t1_ref[...]
    dis = t1[:, 15:16]
    stot = t1 + a_ref[...] + b_ref[...]
    h1 = jnp.dot(stot, w1_ref[...], preferred_element_type=jnp.float32)
    out1 = jnp.maximum(h1 * dis + b1_ref[...], 0.0)
    h2 = jnp.dot(out1, w2_ref[...], preferred_element_type=jnp.float32)
    t2_ref[...] = h2 * dis


def _tc2(t1, s1a, s1b, w1p, b1, w2):
    blk = pl.BlockSpec((BN, 16), lambda i: (i, 0))
    return pl.pallas_call(
        _tc2_body,
        grid=(NB,),
        in_specs=[
            blk, blk, blk,
            pl.BlockSpec((16, 32), lambda i: (0, 0)),
            pl.BlockSpec((1, 32), lambda i: (0, 0)),
            pl.BlockSpec((32, 16), lambda i: (0, 0)),
        ],
        out_specs=pl.BlockSpec((BN, 16), lambda i: (i, 0)),
        out_shape=jax.ShapeDtypeStruct((NP, 16), jnp.float32),
    )(t1, s1a, s1b, w1p, b1.reshape(1, 32), w2)


def _tc3_body(t2_ref, a_ref, b_ref, d0_ref, d1_ref, b2_ref, wf_ref, bf_ref,
              y_ref):
    deg = d0_ref[...] + d1_ref[...] + 1.0
    dis = lax.rsqrt(deg)
    stot = t2_ref[...] + a_ref[...] + b_ref[...]
    out2 = jnp.maximum(stot * dis + b2_ref[...], 0.0)
    y_ref[...] = jnp.dot(out2, wf_ref[...],
                         preferred_element_type=jnp.float32) + bf_ref[...]


def _tc3(t2, s2a, s2b, deg0, deg1, b2, wf, bf):
    blk = pl.BlockSpec((BN, 16), lambda i: (i, 0))
    return pl.pallas_call(
        _tc3_body,
        grid=(NB,),
        in_specs=[
            blk, blk, blk, blk, blk,
            pl.BlockSpec((1, 16), lambda i: (0, 0)),
            pl.BlockSpec((16, 1), lambda i: (0, 0)),
            pl.BlockSpec((1, 1), lambda i: (0, 0)),
        ],
        out_specs=pl.BlockSpec((BN, 1), lambda i: (i, 0)),
        out_shape=jax.ShapeDtypeStruct((N, 1), jnp.float32),
    )(t2, s2a, s2b, deg0, deg1, b2.reshape(1, 16), wf, bf.reshape(1, 1))


def kernel(x, edge_index, W1, b1, W2, b2, Wf, bf):
    src = edge_index[0].astype(jnp.int32)
    dst = edge_index[1].astype(jnp.int32)
    npad = EP - E
    pad_idx = N + (jnp.arange(npad, dtype=jnp.int32) % PADROWS)
    src_p = jnp.concatenate([src, pad_idx])
    dst_p = jnp.concatenate([dst, pad_idx])

    x16 = jnp.zeros((NP, 16), jnp.float32)
    x16 = x16.at[:N, :3].set(x.astype(jnp.float32))
    x16 = x16.at[:, 15].set(1.0)

    w1p = jnp.zeros((16, 32), jnp.float32).at[:3, :].set(W1)

    deg_k, prop_k = _sc_kernels()
    deg0, deg1 = deg_k(dst_p)
    t1 = _tc1(deg0, deg1, x16)
    s1a, s1b = prop_k(t1, src_p, dst_p)
    t2 = _tc2(t1, s1a, s1b, w1p, b1, W2)
    s2a, s2b = prop_k(t2, src_p, dst_p)
    return _tc3(t2, s2a, s2b, deg0, deg1, b2, Wf, bf)


# speedup vs baseline: 104.9106x; 1.6655x over previous
import functools

import jax
import jax.numpy as jnp
from jax import lax
from jax.experimental import pallas as pl
from jax.experimental.pallas import tpu as pltpu
from jax.experimental.pallas import tpu_sc as plsc

N = 100000
E = 3200000
NC, NS = 2, 16
NW = NC * NS
GROUP = 512
SUB = 128
NSUB = GROUP // SUB
EP = ((E + NW * GROUP - 1) // (NW * GROUP)) * (NW * GROUP)
EPW = EP // NW
G = EPW // GROUP
NP = 100352
DRAIN = NP // NS
PADROWS = NP - N
NPF = NP * 16 // 128
BM = 1792
NB = NPF // BM

def _chunks(total, size):
    out, off = [], 0
    while off < total:
        c = min(size, total - off)
        out.append((off, c))
        off += c
    return out


def _deg_body(dst_hbm, out0, out1, didx, vals, acc,
              sem_in0, sem_in1, sem_s0, sem_s1):
    c = lax.axis_index("c")
    s = lax.axis_index("s")
    wid = s * NC + c
    base = wid * EPW
    tile0 = s * DRAIN
    sem_in = (sem_in0, sem_in1)
    sem_s = (sem_s0, sem_s1)

    @pl.loop(0, GROUP)
    def _(i):
        vals[i, :] = jnp.zeros((16,), jnp.float32)

    for off, ln in _chunks(DRAIN, GROUP):
        pltpu.sync_copy(vals.at[pl.ds(0, ln), :],
                        acc.at[pl.ds(tile0 + off, ln), :])

    @pl.loop(0, SUB)
    def _(i):
        vals[i, :] = jnp.ones((16,), jnp.float32)

    plsc.subcore_barrier()

    def in_descs(g, sl):
        e0 = (base + g * GROUP) % EP
        return [pltpu.make_async_copy(dst_hbm.at[pl.ds(e0 + j * SUB, SUB)],
                                      didx.at[sl, j], sem_in[sl])
                for j in range(NSUB)]

    def fire_sc(sl):
        for j in range(NSUB):
            pltpu.async_copy(vals.at[pl.ds(0, SUB), :],
                             acc.at[didx.at[sl, j]], sem_s[sl], add=True)

    def wait_sc(sl):
        for j in range(NSUB):
            pltpu.make_async_copy(vals.at[pl.ds(0, SUB), :],
                                  acc.at[didx.at[sl, j]], sem_s[sl]).wait()

    def fire(descs):
        for d in descs:
            d.start()

    def wait(descs):
        for d in descs:
            d.wait()

    def body(g, first):
        wait(in_descs(g, 0))
        if not first:
            wait_sc(1)
        fire(in_descs(g + 1, 1))
        fire_sc(0)
        wait(in_descs(g + 1, 1))
        wait_sc(0)
        fire(in_descs(g + 2, 0))
        fire_sc(1)

    fire(in_descs(0, 0))
    body(0, True)

    @pl.loop(2, G, step=2)
    def _(g):
        body(g, False)

    wait_sc(1)
    wait(in_descs(G, 0))

    plsc.subcore_barrier()

    @pl.when(c == 0)
    def _():
        pltpu.sync_copy(acc.at[pl.ds(tile0, DRAIN), :],
                        out0.at[pl.ds(tile0, DRAIN), :])

    @pl.when(c == 1)
    def _():
        pltpu.sync_copy(acc.at[pl.ds(tile0, DRAIN), :],
                        out1.at[pl.ds(tile0, DRAIN), :])


def _prop_body(table, src_hbm, dst_hbm, out0, out1,
               sidx, didx, rows, acc, sem_in0, sem_in1, sem_g0, sem_g1,
               sem_s0, sem_s1):
    c = lax.axis_index("c")
    s = lax.axis_index("s")
    wid = s * NC + c
    base = wid * EPW
    tile0 = s * DRAIN
    sem_in = (sem_in0, sem_in1)
    sem_g = (sem_g0, sem_g1)
    sem_s = (sem_s0, sem_s1)

    @pl.loop(0, GROUP)
    def _(i):
        rows[0, i, :] = jnp.zeros((16,), jnp.float32)

    for off, ln in _chunks(DRAIN, GROUP):
        pltpu.sync_copy(rows.at[0, pl.ds(0, ln), :],
                        acc.at[pl.ds(tile0 + off, ln), :])
    plsc.subcore_barrier()

    def in_descs(g, sl):
        e0 = (base + g * GROUP) % EP
        d = [pltpu.make_async_copy(src_hbm.at[pl.ds(e0, GROUP)],
                                   sidx.at[sl], sem_in[sl])]
        d += [pltpu.make_async_copy(dst_hbm.at[pl.ds(e0 + j * SUB, SUB)],
                                    didx.at[sl, j], sem_in[sl])
              for j in range(NSUB)]
        return d

    def ga_descs(g, sl):
        return [pltpu.make_async_copy(
            table.at[sidx.at[sl].at[pl.ds(j * SUB, SUB)]],
            rows.at[sl, pl.ds(j * SUB, SUB), :], sem_g[sl])
            for j in range(NSUB)]

    def fire_sc(g, sl):
        for j in range(NSUB):
            pltpu.async_copy(rows.at[sl, pl.ds(j * SUB, SUB), :],
                             acc.at[didx.at[sl, j]], sem_s[sl], add=True)

    def wait_sc(g, sl):
        for j in range(NSUB):
            pltpu.make_async_copy(rows.at[sl, pl.ds(j * SUB, SUB), :],
                                  acc.at[didx.at[sl, j]], sem_s[sl]).wait()

    def fire(descs):
        for d in descs:
            d.start()

    def wait(descs):
        for d in descs:
            d.wait()

    def body(g, first):
        wait(in_descs(g, 0))
        fire(ga_descs(g, 0))
        if not first:
            wait_sc(g - 1, 1)
        fire(in_descs(g + 1, 1))
        wait(ga_descs(g, 0))
        fire_sc(g, 0)
        wait(in_descs(g + 1, 1))
        fire(ga_descs(g + 1, 1))
        wait_sc(g, 0)
        fire(in_descs(g + 2, 0))
        wait(ga_descs(g + 1, 1))
        fire_sc(g + 1, 1)

    fire(in_descs(0, 0))
    body(0, True)

    @pl.loop(2, G, step=2)
    def _(g):
        body(g, False)

    wait_sc(G - 1, 1)
    wait(in_descs(G, 0))

    plsc.subcore_barrier()

    @pl.when(c == 0)
    def _():
        pltpu.sync_copy(acc.at[pl.ds(tile0, DRAIN), :],
                        out0.at[pl.ds(tile0, DRAIN), :])

    @pl.when(c == 1)
    def _():
        pltpu.sync_copy(acc.at[pl.ds(tile0, DRAIN), :],
                        out1.at[pl.ds(tile0, DRAIN), :])


@functools.lru_cache(maxsize=None)
def _sc_kernels():
    mesh = plsc.VectorSubcoreMesh(core_axis_name="c", subcore_axis_name="s",
                                  num_cores=NC, num_subcores=NS)
    params = pltpu.CompilerParams(use_tc_tiling_on_sc=False)
    deg = pl.kernel(
        _deg_body,
        compiler_params=params,
        out_type=[jax.ShapeDtypeStruct((NP, 16), jnp.float32),
                  jax.ShapeDtypeStruct((NP, 16), jnp.float32)],
        mesh=mesh,
        scratch_types=[
            pltpu.VMEM((2, NSUB, SUB), jnp.int32),
            pltpu.VMEM((GROUP, 16), jnp.float32),
            pltpu.VMEM_SHARED((NP, 16), jnp.float32),
            pltpu.SemaphoreType.DMA,
            pltpu.SemaphoreType.DMA,
            pltpu.SemaphoreType.DMA,
            pltpu.SemaphoreType.DMA,
        ],
    )
    prop = pl.kernel(
        _prop_body,
        compiler_params=params,
        out_type=[jax.ShapeDtypeStruct((NP, 16), jnp.float32),
                  jax.ShapeDtypeStruct((NP, 16), jnp.float32)],
        mesh=mesh,
        scratch_types=[
            pltpu.VMEM((2, GROUP), jnp.int32),
            pltpu.VMEM((2, NSUB, SUB), jnp.int32),
            pltpu.VMEM((2, GROUP, 16), jnp.float32),
            pltpu.VMEM_SHARED((NP, 16), jnp.float32),
            pltpu.SemaphoreType.DMA,
            pltpu.SemaphoreType.DMA,
            pltpu.SemaphoreType.DMA,
            pltpu.SemaphoreType.DMA,
            pltpu.SemaphoreType.DMA,
            pltpu.SemaphoreType.DMA,
        ],
    )
    return deg, prop


_flat = pl.BlockSpec((BM, 128), lambda i: (i, 0))


def _tc1_body(d0_ref, d1_ref, x_ref, t1_ref):
    dis = lax.rsqrt(d0_ref[...] + d1_ref[...] + 1.0)
    t1_ref[...] = x_ref[...] * dis


def _tc1(d0f, d1f, x16f):
    return pl.pallas_call(
        _tc1_body,
        grid=(NB,),
        in_specs=[_flat, _flat, _flat],
        out_specs=_flat,
        out_shape=jax.ShapeDtypeStruct((NPF, 128), jnp.float32),
    )(d0f, d1f, x16f)


def _tc2_body(t1_ref, a_ref, b_ref, d0_ref, d1_ref, w1_ref, b1_ref, w2_ref,
              t2_ref):
    dis = lax.rsqrt(d0_ref[...] + d1_ref[...] + 1.0)
    sd = (t1_ref[...] + a_ref[...] + b_ref[...]) * dis
    h1 = jnp.dot(sd, w1_ref[...], preferred_element_type=jnp.float32)
    out1 = jnp.maximum(h1 + b1_ref[...], 0.0)
    h2 = jnp.dot(out1, w2_ref[...], preferred_element_type=jnp.float32)
    t2_ref[...] = h2 * dis


def _tc2(t1f, s1af, s1bf, d0f, d1f, w1bd, b1t, w2bd):
    return pl.pallas_call(
        _tc2_body,
        grid=(NB,),
        in_specs=[
            _flat, _flat, _flat, _flat, _flat,
            pl.BlockSpec((128, 256), lambda i: (0, 0)),
            pl.BlockSpec((1, 256), lambda i: (0, 0)),
            pl.BlockSpec((256, 128), lambda i: (0, 0)),
        ],
        out_specs=_flat,
        out_shape=jax.ShapeDtypeStruct((NPF, 128), jnp.float32),
    )(t1f, s1af, s1bf, d0f, d1f, w1bd, b1t, w2bd)


def _tc3_body(t2_ref, a_ref, b_ref, d0_ref, d1_ref, b2_ref, wf_ref, bf_ref,
              y_ref):
    dis = lax.rsqrt(d0_ref[...] + d1_ref[...] + 1.0)
    stot = (t2_ref[...] + a_ref[...] + b_ref[...]) * dis
    out2 = jnp.maximum(stot + b2_ref[...], 0.0)
    y_ref[...] = jnp.dot(out2, wf_ref[...],
                         preferred_element_type=jnp.float32) + bf_ref[...]


def _tc3(t2f, s2af, s2bf, d0f, d1f, b2t, wfbd, bf):
    return pl.pallas_call(
        _tc3_body,
        grid=(NB,),
        in_specs=[
            _flat, _flat, _flat, _flat, _flat,
            pl.BlockSpec((1, 128), lambda i: (0, 0)),
            pl.BlockSpec((128, 8), lambda i: (0, 0)),
            pl.BlockSpec((1, 1), lambda i: (0, 0)),
        ],
        out_specs=pl.BlockSpec((BM, 8), lambda i: (i, 0)),
        out_shape=jax.ShapeDtypeStruct((NPF, 8), jnp.float32),
    )(t2f, s2af, s2bf, d0f, d1f, b2t, wfbd, bf.reshape(1, 1))


def _block_diag(w, k):
    r, c = w.shape
    out = jnp.zeros((r * k, c * k), w.dtype)
    for i in range(k):
        out = out.at[i * r:(i + 1) * r, i * c:(i + 1) * c].set(w)
    return out


def kernel(x, edge_index, W1, b1, W2, b2, Wf, bf):
    src = edge_index[0].astype(jnp.int32)
    dst = edge_index[1].astype(jnp.int32)
    npad = EP - E
    pad_idx = N + (jnp.arange(npad, dtype=jnp.int32) % PADROWS)
    src_p = jnp.concatenate([src, pad_idx])
    dst_p = jnp.concatenate([dst, pad_idx])

    x16f = jnp.pad(x.astype(jnp.float32),
                   ((0, NP - N), (0, 13))).reshape(NPF, 128)

    w1p = jnp.zeros((16, 32), jnp.float32).at[:3, :].set(W1)
    w1bd = _block_diag(w1p, 8)
    w2bd = _block_diag(W2, 8)
    wfbd = _block_diag(Wf, 8)
    b1t = jnp.tile(b1, 8).reshape(1, 256)
    b2t = jnp.tile(b2, 8).reshape(1, 128)

    deg_k, prop_k = _sc_kernels()
    deg0, deg1 = deg_k(dst_p)
    d0f = deg0.reshape(NPF, 128)
    d1f = deg1.reshape(NPF, 128)
    t1f = _tc1(d0f, d1f, x16f)
    s1a, s1b = prop_k(t1f.reshape(NP, 16), src_p, dst_p)
    t2f = _tc2(t1f, s1a.reshape(NPF, 128), s1b.reshape(NPF, 128),
               d0f, d1f, w1bd, b1t, w2bd)
    s2a, s2b = prop_k(t2f.reshape(NP, 16), src_p, dst_p)
    y8 = _tc3(t2f, s2a.reshape(NPF, 128), s2b.reshape(NPF, 128),
              d0f, d1f, b2t, wfbd, bf)
    return y8.reshape(NP, 1)[:N]


# speedup vs baseline: 106.7420x; 1.0175x over previous
import functools

import jax
import jax.numpy as jnp
from jax import lax
from jax.experimental import pallas as pl
from jax.experimental.pallas import tpu as pltpu
from jax.experimental.pallas import tpu_sc as plsc

N = 100000
E = 3200000
NC, NS = 2, 16
NW = NC * NS
GROUP = 512
SUB = 128
NSUB = GROUP // SUB
EPW = E // NW
G = -(-EPW // GROUP)
TAIL = EPW - (G - 1) * GROUP
NP = 100352
DRAIN = NP // NS
PADROWS = NP - N
NPF = NP * 16 // 128
BM = 1792
NB = NPF // BM

def _chunks(total, size):
    out, off = [], 0
    while off < total:
        c = min(size, total - off)
        out.append((off, c))
        off += c
    return out


def _pad_vec(m):
    return N + m * 16 + lax.iota(jnp.int32, 16)


def _deg_body(ei_hbm, out0, out1, didx, vals, acc,
              sem_in0, sem_in1, sem_s0, sem_s1):
    c = lax.axis_index("c")
    s = lax.axis_index("s")
    wid = s * NC + c
    base = wid * EPW
    tile0 = s * DRAIN
    sem_in = (sem_in0, sem_in1)
    sem_s = (sem_s0, sem_s1)
    dst_hbm = ei_hbm.at[1]

    @pl.loop(0, GROUP)
    def _(i):
        vals[i, :] = jnp.zeros((16,), jnp.float32)

    for off, ln in _chunks(DRAIN, GROUP):
        pltpu.sync_copy(vals.at[pl.ds(0, ln), :],
                        acc.at[pl.ds(tile0 + off, ln), :])

    @pl.loop(0, SUB)
    def _(i):
        vals[i, :] = jnp.ones((16,), jnp.float32)

    plsc.subcore_barrier()

    def in_descs(g, sl, tail=False):
        e0 = base + g * GROUP
        if not tail:
            return [pltpu.make_async_copy(dst_hbm.at[pl.ds(e0 + j * SUB, SUB)],
                                          didx.at[sl, j], sem_in[sl])
                    for j in range(NSUB)]
        return [pltpu.make_async_copy(dst_hbm.at[pl.ds(e0, SUB)],
                                      didx.at[sl, 0], sem_in[sl]),
                pltpu.make_async_copy(dst_hbm.at[pl.ds(e0 + SUB, TAIL - SUB)],
                                      didx.at[sl, 1].at[pl.ds(0, TAIL - SUB)],
                                      sem_in[sl])]

    def fill_tail(sl):
        m = 0
        for i in range(TAIL - SUB, SUB, 16):
            didx[sl, 1, pl.ds(i, 16)] = _pad_vec(m)
            m += 1
        for j in range(2, NSUB):
            for i in range(0, SUB, 16):
                didx[sl, j, pl.ds(i, 16)] = _pad_vec(m)
                m += 1

    def fire_sc(sl):
        for j in range(NSUB):
            pltpu.async_copy(vals.at[pl.ds(0, SUB), :],
                             acc.at[didx.at[sl, j]], sem_s[sl], add=True)

    def wait_sc(sl):
        for j in range(NSUB):
            pltpu.make_async_copy(vals.at[pl.ds(0, SUB), :],
                                  acc.at[didx.at[sl, j]], sem_s[sl]).wait()

    def fire(descs):
        for d in descs:
            d.start()

    def wait(descs):
        for d in descs:
            d.wait()

    def body(g, first=False, last=False):
        wait(in_descs(g, 0))
        if not first:
            wait_sc(1)
        fire(in_descs(g + 1, 1, tail=last))
        if last:
            fill_tail(1)
        fire_sc(0)
        wait(in_descs(g + 1, 1, tail=last))
        wait_sc(0)
        if not last:
            fire(in_descs(g + 2, 0))
        fire_sc(1)

    fire(in_descs(0, 0))
    body(0, first=True)

    @pl.loop(2, G - 2, step=2)
    def _(g):
        body(g)

    body(G - 2, last=True)
    wait_sc(1)

    plsc.subcore_barrier()

    @pl.when(c == 0)
    def _():
        pltpu.sync_copy(acc.at[pl.ds(tile0, DRAIN), :],
                        out0.at[pl.ds(tile0, DRAIN), :])

    @pl.when(c == 1)
    def _():
        pltpu.sync_copy(acc.at[pl.ds(tile0, DRAIN), :],
                        out1.at[pl.ds(tile0, DRAIN), :])


def _prop_body(table, ei_hbm, out0, out1,
               sidx, didx, rows, acc, sem_in0, sem_in1, sem_g0, sem_g1,
               sem_s0, sem_s1):
    c = lax.axis_index("c")
    s = lax.axis_index("s")
    wid = s * NC + c
    base = wid * EPW
    tile0 = s * DRAIN
    sem_in = (sem_in0, sem_in1)
    sem_g = (sem_g0, sem_g1)
    sem_s = (sem_s0, sem_s1)
    src_hbm = ei_hbm.at[0]
    dst_hbm = ei_hbm.at[1]

    @pl.loop(0, GROUP)
    def _(i):
        rows[0, i, :] = jnp.zeros((16,), jnp.float32)

    for off, ln in _chunks(DRAIN, GROUP):
        pltpu.sync_copy(rows.at[0, pl.ds(0, ln), :],
                        acc.at[pl.ds(tile0 + off, ln), :])
    plsc.subcore_barrier()

    def in_descs(g, sl, tail=False):
        e0 = base + g * GROUP
        if not tail:
            d = [pltpu.make_async_copy(src_hbm.at[pl.ds(e0, GROUP)],
                                       sidx.at[sl], sem_in[sl])]
            d += [pltpu.make_async_copy(dst_hbm.at[pl.ds(e0 + j * SUB, SUB)],
                                        didx.at[sl, j], sem_in[sl])
                  for j in range(NSUB)]
            return d
        return [pltpu.make_async_copy(src_hbm.at[pl.ds(e0, TAIL)],
                                      sidx.at[sl].at[pl.ds(0, TAIL)],
                                      sem_in[sl]),
                pltpu.make_async_copy(dst_hbm.at[pl.ds(e0, SUB)],
                                      didx.at[sl, 0], sem_in[sl]),
                pltpu.make_async_copy(dst_hbm.at[pl.ds(e0 + SUB, TAIL - SUB)],
                                      didx.at[sl, 1].at[pl.ds(0, TAIL - SUB)],
                                      sem_in[sl])]

    def fill_tail(sl):
        m = 0
        for i in range(TAIL, GROUP, 16):
            sidx[sl, pl.ds(i, 16)] = _pad_vec(m % 22)
            m += 1
        m = 0
        for i in range(TAIL - SUB, SUB, 16):
            didx[sl, 1, pl.ds(i, 16)] = _pad_vec(m)
            m += 1
        for j in range(2, NSUB):
            for i in range(0, SUB, 16):
                didx[sl, j, pl.ds(i, 16)] = _pad_vec(m)
                m += 1

    def ga_descs(g, sl):
        return [pltpu.make_async_copy(
            table.at[sidx.at[sl].at[pl.ds(j * SUB, SUB)]],
            rows.at[sl, pl.ds(j * SUB, SUB), :], sem_g[sl])
            for j in range(NSUB)]

    def fire_sc(g, sl):
        for j in range(NSUB):
            pltpu.async_copy(rows.at[sl, pl.ds(j * SUB, SUB), :],
                             acc.at[didx.at[sl, j]], sem_s[sl], add=True)

    def wait_sc(g, sl):
        for j in range(NSUB):
            pltpu.make_async_copy(rows.at[sl, pl.ds(j * SUB, SUB), :],
                                  acc.at[didx.at[sl, j]], sem_s[sl]).wait()

    def fire(descs):
        for d in descs:
            d.start()

    def wait(descs):
        for d in descs:
            d.wait()

    def body(g, first=False, last=False):
        wait(in_descs(g, 0))
        fire(ga_descs(g, 0))
        if not first:
            wait_sc(g - 1, 1)
        fire(in_descs(g + 1, 1, tail=last))
        if last:
            fill_tail(1)
        wait(ga_descs(g, 0))
        fire_sc(g, 0)
        wait(in_descs(g + 1, 1, tail=last))
        fire(ga_descs(g + 1, 1))
        wait_sc(g, 0)
        if not last:
            fire(in_descs(g + 2, 0))
        wait(ga_descs(g + 1, 1))
        fire_sc(g + 1, 1)

    fire(in_descs(0, 0))
    body(0, first=True)

    @pl.loop(2, G - 2, step=2)
    def _(g):
        body(g)

    body(G - 2, last=True)
    wait_sc(G - 1, 1)

    plsc.subcore_barrier()

    @pl.when(c == 0)
    def _():
        pltpu.sync_copy(acc.at[pl.ds(tile0, DRAIN), :],
                        out0.at[pl.ds(tile0, DRAIN), :])

    @pl.when(c == 1)
    def _():
        pltpu.sync_copy(acc.at[pl.ds(tile0, DRAIN), :],
                        out1.at[pl.ds(tile0, DRAIN), :])


@functools.lru_cache(maxsize=None)
def _sc_kernels():
    mesh = plsc.VectorSubcoreMesh(core_axis_name="c", subcore_axis_name="s",
                                  num_cores=NC, num_subcores=NS)
    params = pltpu.CompilerParams(use_tc_tiling_on_sc=False)
    deg = pl.kernel(
        _deg_body,
        compiler_params=params,
        out_type=[jax.ShapeDtypeStruct((NP, 16), jnp.float32),
                  jax.ShapeDtypeStruct((NP, 16), jnp.float32)],
        mesh=mesh,
        scratch_types=[
            pltpu.VMEM((2, NSUB, SUB), jnp.int32),
            pltpu.VMEM((GROUP, 16), jnp.float32),
            pltpu.VMEM_SHARED((NP, 16), jnp.float32),
            pltpu.SemaphoreType.DMA,
            pltpu.SemaphoreType.DMA,
            pltpu.SemaphoreType.DMA,
            pltpu.SemaphoreType.DMA,
        ],
    )
    prop = pl.kernel(
        _prop_body,
        compiler_params=params,
        out_type=[jax.ShapeDtypeStruct((NP, 16), jnp.float32),
                  jax.ShapeDtypeStruct((NP, 16), jnp.float32)],
        mesh=mesh,
        scratch_types=[
            pltpu.VMEM((2, GROUP), jnp.int32),
            pltpu.VMEM((2, NSUB, SUB), jnp.int32),
            pltpu.VMEM((2, GROUP, 16), jnp.float32),
            pltpu.VMEM_SHARED((NP, 16), jnp.float32),
            pltpu.SemaphoreType.DMA,
            pltpu.SemaphoreType.DMA,
            pltpu.SemaphoreType.DMA,
            pltpu.SemaphoreType.DMA,
            pltpu.SemaphoreType.DMA,
            pltpu.SemaphoreType.DMA,
        ],
    )
    return deg, prop


_flat = pl.BlockSpec((BM, 128), lambda i: (i, 0))


def _tc1_body(d0_ref, d1_ref, x_ref, t1_ref):
    dis = lax.rsqrt(d0_ref[...] + d1_ref[...] + 1.0)
    t1_ref[...] = x_ref[...] * dis


def _tc1(d0f, d1f, x16f):
    return pl.pallas_call(
        _tc1_body,
        grid=(NB,),
        in_specs=[_flat, _flat, _flat],
        out_specs=_flat,
        out_shape=jax.ShapeDtypeStruct((NPF, 128), jnp.float32),
    )(d0f, d1f, x16f)


def _tc2_body(t1_ref, a_ref, b_ref, d0_ref, d1_ref, w1_ref, b1_ref, w2_ref,
              t2_ref):
    dis = lax.rsqrt(d0_ref[...] + d1_ref[...] + 1.0)
    sd = (t1_ref[...] + a_ref[...] + b_ref[...]) * dis
    h1 = jnp.dot(sd, w1_ref[...], preferred_element_type=jnp.float32)
    out1 = jnp.maximum(h1 + b1_ref[...], 0.0)
    h2 = jnp.dot(out1, w2_ref[...], preferred_element_type=jnp.float32)
    t2_ref[...] = h2 * dis


def _tc2(t1f, s1af, s1bf, d0f, d1f, w1bd, b1t, w2bd):
    return pl.pallas_call(
        _tc2_body,
        grid=(NB,),
        in_specs=[
            _flat, _flat, _flat, _flat, _flat,
            pl.BlockSpec((128, 256), lambda i: (0, 0)),
            pl.BlockSpec((1, 256), lambda i: (0, 0)),
            pl.BlockSpec((256, 128), lambda i: (0, 0)),
        ],
        out_specs=_flat,
        out_shape=jax.ShapeDtypeStruct((NPF, 128), jnp.float32),
    )(t1f, s1af, s1bf, d0f, d1f, w1bd, b1t, w2bd)


def _tc3_body(t2_ref, a_ref, b_ref, d0_ref, d1_ref, b2_ref, wf_ref, bf_ref,
              y_ref):
    dis = lax.rsqrt(d0_ref[...] + d1_ref[...] + 1.0)
    stot = (t2_ref[...] + a_ref[...] + b_ref[...]) * dis
    out2 = jnp.maximum(stot + b2_ref[...], 0.0)
    y_ref[...] = jnp.dot(out2, wf_ref[...],
                         preferred_element_type=jnp.float32) + bf_ref[...]


def _tc3(t2f, s2af, s2bf, d0f, d1f, b2t, wfbd, bf):
    return pl.pallas_call(
        _tc3_body,
        grid=(NB,),
        in_specs=[
            _flat, _flat, _flat, _flat, _flat,
            pl.BlockSpec((1, 128), lambda i: (0, 0)),
            pl.BlockSpec((128, 8), lambda i: (0, 0)),
            pl.BlockSpec((1, 1), lambda i: (0, 0)),
        ],
        out_specs=pl.BlockSpec((BM, 8), lambda i: (i, 0)),
        out_shape=jax.ShapeDtypeStruct((NPF, 8), jnp.float32),
    )(t2f, s2af, s2bf, d0f, d1f, b2t, wfbd, bf.reshape(1, 1))


def _block_diag(w, k):
    r, c = w.shape
    out = jnp.zeros((r * k, c * k), w.dtype)
    for i in range(k):
        out = out.at[i * r:(i + 1) * r, i * c:(i + 1) * c].set(w)
    return out


def kernel(x, edge_index, W1, b1, W2, b2, Wf, bf):
    ei = edge_index.astype(jnp.int32)

    x16f = jnp.pad(x.astype(jnp.float32),
                   ((0, NP - N), (0, 13))).reshape(NPF, 128)

    w1p = jnp.zeros((16, 32), jnp.float32).at[:3, :].set(W1)
    w1bd = _block_diag(w1p, 8)
    w2bd = _block_diag(W2, 8)
    wfbd = _block_diag(Wf, 8)
    b1t = jnp.tile(b1, 8).reshape(1, 256)
    b2t = jnp.tile(b2, 8).reshape(1, 128)

    deg_k, prop_k = _sc_kernels()
    deg0, deg1 = deg_k(ei)
    d0f = deg0.reshape(NPF, 128)
    d1f = deg1.reshape(NPF, 128)
    t1f = _tc1(d0f, d1f, x16f)
    s1a, s1b = prop_k(t1f.reshape(NP, 16), ei)
    t2f = _tc2(t1f, s1a.reshape(NPF, 128), s1b.reshape(NPF, 128),
               d0f, d1f, w1bd, b1t, w2bd)
    s2a, s2b = prop_k(t2f.reshape(NP, 16), ei)
    y8 = _tc3(t2f, s2a.reshape(NPF, 128), s2b.reshape(NPF, 128),
              d0f, d1f, b2t, wfbd, bf)
    return y8.reshape(NP, 1)[:N]


# speedup vs baseline: 114.5497x; 1.0731x over previous
import functools

import jax
import jax.numpy as jnp
from jax import lax
from jax.experimental import pallas as pl
from jax.experimental.pallas import tpu as pltpu
from jax.experimental.pallas import tpu_sc as plsc

N = 100000
E = 3200000
NC, NS = 2, 16
NW = NC * NS
GROUP = 512
SUB = 128
NSUB = GROUP // SUB
EPW = E // NW
G = -(-EPW // GROUP)
TAIL = EPW - (G - 1) * GROUP
NP = 100352
DRAIN = NP // NS
PADROWS = NP - N
NPF = NP * 16 // 128
BM = 1792
NB = NPF // BM

def _chunks(total, size):
    out, off = [], 0
    while off < total:
        c = min(size, total - off)
        out.append((off, c))
        off += c
    return out


def _pad_vec(m):
    return N + m * 16 + lax.iota(jnp.int32, 16)


def _deg_body(ei_hbm, out0, out1, didx, vals, acc,
              sem_in0, sem_in1, sem_s0, sem_s1):
    c = lax.axis_index("c")
    s = lax.axis_index("s")
    wid = s * NC + c
    base = wid * EPW
    tile0 = s * DRAIN
    sem_in = (sem_in0, sem_in1)
    sem_s = (sem_s0, sem_s1)
    dst_hbm = ei_hbm.at[1]

    @pl.loop(0, GROUP)
    def _(i):
        vals[i, :] = jnp.zeros((16,), jnp.float32)

    for off, ln in _chunks(DRAIN, GROUP):
        pltpu.sync_copy(vals.at[pl.ds(0, ln), :],
                        acc.at[pl.ds(tile0 + off, ln), :])

    @pl.loop(0, SUB)
    def _(i):
        vals[i, :] = jnp.ones((16,), jnp.float32)

    plsc.subcore_barrier()

    def in_descs(g, sl, tail=False):
        e0 = base + g * GROUP
        if not tail:
            return [pltpu.make_async_copy(dst_hbm.at[pl.ds(e0 + j * SUB, SUB)],
                                          didx.at[sl, j], sem_in[sl])
                    for j in range(NSUB)]
        return [pltpu.make_async_copy(dst_hbm.at[pl.ds(e0, SUB)],
                                      didx.at[sl, 0], sem_in[sl]),
                pltpu.make_async_copy(dst_hbm.at[pl.ds(e0 + SUB, TAIL - SUB)],
                                      didx.at[sl, 1].at[pl.ds(0, TAIL - SUB)],
                                      sem_in[sl])]

    def fill_tail(sl):
        m = 0
        for i in range(TAIL - SUB, SUB, 16):
            didx[sl, 1, pl.ds(i, 16)] = _pad_vec(m)
            m += 1
        for j in range(2, NSUB):
            for i in range(0, SUB, 16):
                didx[sl, j, pl.ds(i, 16)] = _pad_vec(m)
                m += 1

    def fire_sc(sl):
        for j in range(NSUB):
            pltpu.async_copy(vals.at[pl.ds(0, SUB), :],
                             acc.at[didx.at[sl, j]], sem_s[sl], add=True)

    def wait_sc(sl):
        for j in range(NSUB):
            pltpu.make_async_copy(vals.at[pl.ds(0, SUB), :],
                                  acc.at[didx.at[sl, j]], sem_s[sl]).wait()

    def fire(descs):
        for d in descs:
            d.start()

    def wait(descs):
        for d in descs:
            d.wait()

    def body(g, first=False, last=False):
        wait(in_descs(g, 0))
        if not first:
            wait_sc(1)
        fire_sc(0)
        fire(in_descs(g + 1, 1, tail=last))
        if last:
            fill_tail(1)
        wait(in_descs(g + 1, 1, tail=last))
        fire_sc(1)
        wait_sc(0)
        if not last:
            fire(in_descs(g + 2, 0))

    fire(in_descs(0, 0))
    body(0, first=True)

    @pl.loop(2, G - 2, step=2)
    def _(g):
        body(g)

    body(G - 2, last=True)
    wait_sc(1)

    plsc.subcore_barrier()

    @pl.when(c == 0)
    def _():
        pltpu.sync_copy(acc.at[pl.ds(tile0, DRAIN), :],
                        out0.at[pl.ds(tile0, DRAIN), :])

    @pl.when(c == 1)
    def _():
        pltpu.sync_copy(acc.at[pl.ds(tile0, DRAIN), :],
                        out1.at[pl.ds(tile0, DRAIN), :])


def _prop_body(table, ei_hbm, out0, out1,
               sidx, didx, rows, acc, sem_in0, sem_in1, sem_g0, sem_g1,
               sem_s0, sem_s1):
    c = lax.axis_index("c")
    s = lax.axis_index("s")
    wid = s * NC + c
    base = wid * EPW
    tile0 = s * DRAIN
    sem_in = (sem_in0, sem_in1)
    sem_g = (sem_g0, sem_g1)
    sem_s = (sem_s0, sem_s1)
    src_hbm = ei_hbm.at[0]
    dst_hbm = ei_hbm.at[1]

    @pl.loop(0, GROUP)
    def _(i):
        rows[0, i, :] = jnp.zeros((16,), jnp.float32)

    for off, ln in _chunks(DRAIN, GROUP):
        pltpu.sync_copy(rows.at[0, pl.ds(0, ln), :],
                        acc.at[pl.ds(tile0 + off, ln), :])
    plsc.subcore_barrier()

    def in_descs(g, sl, tail=False):
        e0 = base + g * GROUP
        if not tail:
            d = [pltpu.make_async_copy(src_hbm.at[pl.ds(e0, GROUP)],
                                       sidx.at[sl], sem_in[sl])]
            d += [pltpu.make_async_copy(dst_hbm.at[pl.ds(e0 + j * SUB, SUB)],
                                        didx.at[sl, j], sem_in[sl])
                  for j in range(NSUB)]
            return d
        return [pltpu.make_async_copy(src_hbm.at[pl.ds(e0, TAIL)],
                                      sidx.at[sl].at[pl.ds(0, TAIL)],
                                      sem_in[sl]),
                pltpu.make_async_copy(dst_hbm.at[pl.ds(e0, SUB)],
                                      didx.at[sl, 0], sem_in[sl]),
                pltpu.make_async_copy(dst_hbm.at[pl.ds(e0 + SUB, TAIL - SUB)],
                                      didx.at[sl, 1].at[pl.ds(0, TAIL - SUB)],
                                      sem_in[sl])]

    def fill_tail(sl):
        m = 0
        for i in range(TAIL, GROUP, 16):
            sidx[sl, pl.ds(i, 16)] = _pad_vec(m % 22)
            m += 1
        m = 0
        for i in range(TAIL - SUB, SUB, 16):
            didx[sl, 1, pl.ds(i, 16)] = _pad_vec(m)
            m += 1
        for j in range(2, NSUB):
            for i in range(0, SUB, 16):
                didx[sl, j, pl.ds(i, 16)] = _pad_vec(m)
                m += 1

    def ga_descs(g, sl):
        return [pltpu.make_async_copy(
            table.at[sidx.at[sl].at[pl.ds(j * SUB, SUB)]],
            rows.at[sl, pl.ds(j * SUB, SUB), :], sem_g[sl])
            for j in range(NSUB)]

    def fire_sc(g, sl):
        for j in range(NSUB):
            pltpu.async_copy(rows.at[sl, pl.ds(j * SUB, SUB), :],
                             acc.at[didx.at[sl, j]], sem_s[sl], add=True)

    def wait_sc(g, sl):
        for j in range(NSUB):
            pltpu.make_async_copy(rows.at[sl, pl.ds(j * SUB, SUB), :],
                                  acc.at[didx.at[sl, j]], sem_s[sl]).wait()

    def fire(descs):
        for d in descs:
            d.start()

    def wait(descs):
        for d in descs:
            d.wait()

    def body(g, first=False, last=False):
        wait(in_descs(g, 0))
        if not first:
            wait_sc(g - 1, 1)
        fire(ga_descs(g, 0))
        fire(in_descs(g + 1, 1, tail=last))
        if last:
            fill_tail(1)
        wait(in_descs(g + 1, 1, tail=last))
        fire(ga_descs(g + 1, 1))
        wait(ga_descs(g, 0))
        fire_sc(g, 0)
        wait_sc(g, 0)
        if not last:
            fire(in_descs(g + 2, 0))
        wait(ga_descs(g + 1, 1))
        fire_sc(g + 1, 1)

    fire(in_descs(0, 0))
    body(0, first=True)

    @pl.loop(2, G - 2, step=2)
    def _(g):
        body(g)

    body(G - 2, last=True)
    wait_sc(G - 1, 1)

    plsc.subcore_barrier()

    @pl.when(c == 0)
    def _():
        pltpu.sync_copy(acc.at[pl.ds(tile0, DRAIN), :],
                        out0.at[pl.ds(tile0, DRAIN), :])

    @pl.when(c == 1)
    def _():
        pltpu.sync_copy(acc.at[pl.ds(tile0, DRAIN), :],
                        out1.at[pl.ds(tile0, DRAIN), :])


@functools.lru_cache(maxsize=None)
def _sc_kernels():
    mesh = plsc.VectorSubcoreMesh(core_axis_name="c", subcore_axis_name="s",
                                  num_cores=NC, num_subcores=NS)
    params = pltpu.CompilerParams(use_tc_tiling_on_sc=False)
    deg = pl.kernel(
        _deg_body,
        compiler_params=params,
        out_type=[jax.ShapeDtypeStruct((NP, 16), jnp.float32),
                  jax.ShapeDtypeStruct((NP, 16), jnp.float32)],
        mesh=mesh,
        scratch_types=[
            pltpu.VMEM((2, NSUB, SUB), jnp.int32),
            pltpu.VMEM((GROUP, 16), jnp.float32),
            pltpu.VMEM_SHARED((NP, 16), jnp.float32),
            pltpu.SemaphoreType.DMA,
            pltpu.SemaphoreType.DMA,
            pltpu.SemaphoreType.DMA,
            pltpu.SemaphoreType.DMA,
        ],
    )
    prop = pl.kernel(
        _prop_body,
        compiler_params=params,
        out_type=[jax.ShapeDtypeStruct((NP, 16), jnp.float32),
                  jax.ShapeDtypeStruct((NP, 16), jnp.float32)],
        mesh=mesh,
        scratch_types=[
            pltpu.VMEM((2, GROUP), jnp.int32),
            pltpu.VMEM((2, NSUB, SUB), jnp.int32),
            pltpu.VMEM((2, GROUP, 16), jnp.float32),
            pltpu.VMEM_SHARED((NP, 16), jnp.float32),
            pltpu.SemaphoreType.DMA,
            pltpu.SemaphoreType.DMA,
            pltpu.SemaphoreType.DMA,
            pltpu.SemaphoreType.DMA,
            pltpu.SemaphoreType.DMA,
            pltpu.SemaphoreType.DMA,
        ],
    )
    return deg, prop


_flat = pl.BlockSpec((BM, 128), lambda i: (i, 0))


def _tc1_body(d0_ref, d1_ref, x_ref, t1_ref):
    dis = lax.rsqrt(d0_ref[...] + d1_ref[...] + 1.0)
    t1_ref[...] = x_ref[...] * dis


def _tc1(d0f, d1f, x16f):
    return pl.pallas_call(
        _tc1_body,
        grid=(NB,),
        in_specs=[_flat, _flat, _flat],
        out_specs=_flat,
        out_shape=jax.ShapeDtypeStruct((NPF, 128), jnp.float32),
    )(d0f, d1f, x16f)


def _tc2_body(t1_ref, a_ref, b_ref, d0_ref, d1_ref, w1_ref, b1_ref, w2_ref,
              t2_ref):
    dis = lax.rsqrt(d0_ref[...] + d1_ref[...] + 1.0)
    sd = (t1_ref[...] + a_ref[...] + b_ref[...]) * dis
    h1 = jnp.dot(sd, w1_ref[...], preferred_element_type=jnp.float32)
    out1 = jnp.maximum(h1 + b1_ref[...], 0.0)
    h2 = jnp.dot(out1, w2_ref[...], preferred_element_type=jnp.float32)
    t2_ref[...] = h2 * dis


def _tc2(t1f, s1af, s1bf, d0f, d1f, w1bd, b1t, w2bd):
    return pl.pallas_call(
        _tc2_body,
        grid=(NB,),
        in_specs=[
            _flat, _flat, _flat, _flat, _flat,
            pl.BlockSpec((128, 256), lambda i: (0, 0)),
            pl.BlockSpec((1, 256), lambda i: (0, 0)),
            pl.BlockSpec((256, 128), lambda i: (0, 0)),
        ],
        out_specs=_flat,
        out_shape=jax.ShapeDtypeStruct((NPF, 128), jnp.float32),
    )(t1f, s1af, s1bf, d0f, d1f, w1bd, b1t, w2bd)


def _tc3_body(t2_ref, a_ref, b_ref, d0_ref, d1_ref, b2_ref, wf_ref, bf_ref,
              y_ref):
    dis = lax.rsqrt(d0_ref[...] + d1_ref[...] + 1.0)
    stot = (t2_ref[...] + a_ref[...] + b_ref[...]) * dis
    out2 = jnp.maximum(stot + b2_ref[...], 0.0)
    y_ref[...] = jnp.dot(out2, wf_ref[...],
                         preferred_element_type=jnp.float32) + bf_ref[...]


def _tc3(t2f, s2af, s2bf, d0f, d1f, b2t, wfbd, bf):
    return pl.pallas_call(
        _tc3_body,
        grid=(NB,),
        in_specs=[
            _flat, _flat, _flat, _flat, _flat,
            pl.BlockSpec((1, 128), lambda i: (0, 0)),
            pl.BlockSpec((128, 8), lambda i: (0, 0)),
            pl.BlockSpec((1, 1), lambda i: (0, 0)),
        ],
        out_specs=pl.BlockSpec((BM, 8), lambda i: (i, 0)),
        out_shape=jax.ShapeDtypeStruct((NPF, 8), jnp.float32),
    )(t2f, s2af, s2bf, d0f, d1f, b2t, wfbd, bf.reshape(1, 1))


def _block_diag(w, k):
    r, c = w.shape
    out = jnp.zeros((r * k, c * k), w.dtype)
    for i in range(k):
        out = out.at[i * r:(i + 1) * r, i * c:(i + 1) * c].set(w)
    return out


def kernel(x, edge_index, W1, b1, W2, b2, Wf, bf):
    ei = edge_index.astype(jnp.int32)

    x16f = jnp.pad(x.astype(jnp.float32),
                   ((0, NP - N), (0, 13))).reshape(NPF, 128)

    w1p = jnp.zeros((16, 32), jnp.float32).at[:3, :].set(W1)
    w1bd = _block_diag(w1p, 8)
    w2bd = _block_diag(W2, 8)
    wfbd = _block_diag(Wf, 8)
    b1t = jnp.tile(b1, 8).reshape(1, 256)
    b2t = jnp.tile(b2, 8).reshape(1, 128)

    deg_k, prop_k = _sc_kernels()
    deg0, deg1 = deg_k(ei)
    d0f = deg0.reshape(NPF, 128)
    d1f = deg1.reshape(NPF, 128)
    t1f = _tc1(d0f, d1f, x16f)
    s1a, s1b = prop_k(t1f.reshape(NP, 16), ei)
    t2f = _tc2(t1f, s1a.reshape(NPF, 128), s1b.reshape(NPF, 128),
               d0f, d1f, w1bd, b1t, w2bd)
    s2a, s2b = prop_k(t2f.reshape(NP, 16), ei)
    y8 = _tc3(t2f, s2a.reshape(NPF, 128), s2b.reshape(NPF, 128),
              d0f, d1f, b2t, wfbd, bf)
    return y8.reshape(NP, 1)[:N]
